# baseline (speedup 1.0000x reference)
import functools

import jax
import jax.numpy as jnp
from jax import lax
from jax.experimental import pallas as pl
from jax.experimental.pallas import tpu as pltpu
from jax.experimental.pallas import tpu_sc as plsc

N = 10000
NP = 10240
E = 320000
D = 128
EPS = 1e-5

NC = 2
NS = 16
NW = NC * NS
K = 128
CPT = 80
EP = NW * CPT * K
HCH = 40
RPW = NP // NS
HL = 16

_mesh = plsc.VectorSubcoreMesh(core_axis_name="c", subcore_axis_name="s")


NR = NP // D


@functools.partial(
    pl.kernel,
    out_type=(
        jax.ShapeDtypeStruct((NR, D), jnp.float32),
        jax.ShapeDtypeStruct((NR, D), jnp.float32),
    ),
    mesh=_mesh,
    compiler_params=pltpu.CompilerParams(needs_layout_passes=False),
    scratch_types=dict(
        deg=pltpu.VMEM_SHARED((NR, D), jnp.float32),
        hist=pltpu.VMEM((NR, D), jnp.float32),
        didxf=pltpu.VMEM((CPT * K,), jnp.int32),
        idb=pltpu.VMEM((NR,), jnp.int32),
        msem=pltpu.SemaphoreType.DMA,
    ),
)
def _hist_kernel(dst_hbm, degA_hbm, degB_hbm, *, deg, hist, didxf, idb, msem):
    c = lax.axis_index("c")
    s = lax.axis_index("s")
    wid = c * NS + s
    ebase = wid * CPT * K
    nvec = CPT * K // 16

    pltpu.sync_copy(dst_hbm.at[pl.ds(pl.multiple_of(ebase, 8), CPT * K)],
                    didxf)

    zero = jnp.zeros((16,), jnp.float32)
    ones = jnp.ones((16,), jnp.float32)
    lane = lax.iota(jnp.int32, 16)

    def fill_z(i, _):
        r = i // 8
        j = i % 8
        hist[r, pl.ds(j * 16, 16)] = zero
        return 0
    lax.fori_loop(0, NR * 8, fill_z, 0)

    def fill_id(k, _):
        idb[pl.ds(k * 16, 16)] = lane + k * 16
        return 0
    lax.fori_loop(0, NR // 16, fill_id, 0)

    def vec(i, _):
        idx = didxf[pl.ds(i * 16, 16)]
        hi = lax.shift_right_logical(idx, jnp.full((16,), 7, jnp.int32))
        lo = lax.bitwise_and(idx, jnp.full((16,), 127, jnp.int32))
        plsc.addupdate_scatter(hist, [hi, lo], ones)
        return 0
    lax.fori_loop(0, nvec, vec, 0)

    @pl.when(s == 0)
    def _():
        pltpu.sync_copy(hist, deg)
    plsc.subcore_barrier()

    @pl.when(s > 0)
    def _():
        pltpu.sync_copy(hist, deg.at[idb], add=True)
    plsc.subcore_barrier()

    @pl.when(s == 0)
    def _():
        @pl.when(c == 0)
        def _():
            pltpu.sync_copy(deg, degA_hbm)

        @pl.when(c == 1)
        def _():
            pltpu.sync_copy(deg, degB_hbm)


@functools.partial(
    pl.kernel,
    out_type=(
        jax.ShapeDtypeStruct((NP, D), jnp.float32),
        jax.ShapeDtypeStruct((NP, D), jnp.float32),
    ),
    mesh=_mesh,
    scratch_types=dict(
        acc=pltpu.VMEM_SHARED((NP, D), jnp.float32),
        rows0=pltpu.VMEM((K, D), jnp.float32),
        rows1=pltpu.VMEM((K, D), jnp.float32),
        sidx=pltpu.VMEM((HCH, K), jnp.int32),
        didx=pltpu.VMEM((HCH, K), jnp.int32),
        gsem0=pltpu.SemaphoreType.DMA,
        gsem1=pltpu.SemaphoreType.DMA,
        ssem0=pltpu.SemaphoreType.DMA,
        ssem1=pltpu.SemaphoreType.DMA,
        dsem0=pltpu.SemaphoreType.DMA,
        dsem1=pltpu.SemaphoreType.DMA,
    ),
)
def _scatter_kernel(g_hbm, src_hbm, dst_hbm, accA_hbm, accB_hbm, *,
                    acc, rows0, rows1, sidx, didx,
                    gsem0, gsem1, ssem0, ssem1, dsem0, dsem1):
    c = lax.axis_index("c")
    s = lax.axis_index("s")
    wid = c * NS + s
    cbase = wid * CPT
    row0 = s * RPW

    pltpu.sync_copy(g_hbm.at[pl.ds(row0, RPW)], acc.at[pl.ds(row0, RPW)])
    plsc.subcore_barrier()

    def pair(p, _):
        pltpu.async_copy(g_hbm.at[sidx.at[2 * p]], rows0, gsem0).wait()

        @pl.when(p > 0)
        def _():
            pltpu.make_async_copy(rows1, acc.at[didx.at[2 * p - 1]],
                                  ssem1).wait()

        s0 = pltpu.async_copy(rows0, acc.at[didx.at[2 * p]], ssem0, add=True)
        pltpu.async_copy(g_hbm.at[sidx.at[2 * p + 1]], rows1, gsem1).wait()
        s0.wait()
        pltpu.async_copy(rows1, acc.at[didx.at[2 * p + 1]], ssem1, add=True)
        return 0

    for h in range(2):
        if h > 0:
            pltpu.make_async_copy(rows1, acc.at[didx.at[HCH - 1]],
                                  ssem1).wait()
        pltpu.sync_copy(src_hbm.at[pl.ds(cbase + h * HCH, HCH)], sidx)
        pltpu.sync_copy(dst_hbm.at[pl.ds(cbase + h * HCH, HCH)], didx)
        lax.fori_loop(0, HCH // 2, pair, 0)
    pltpu.make_async_copy(rows1, acc.at[didx.at[HCH - 1]], ssem1).wait()

    plsc.subcore_barrier()

    @pl.when(c == 0)
    def _():
        pltpu.sync_copy(acc.at[pl.ds(row0, RPW)], accA_hbm.at[pl.ds(row0, RPW)])

    @pl.when(c == 1)
    def _():
        pltpu.sync_copy(acc.at[pl.ds(row0, RPW)], accB_hbm.at[pl.ds(row0, RPW)])


_MM_BLK = 1024
_BN_BLK = 1000


def _mm_body(x_ref, w_ref, da_ref, db_ref, g_ref):
    deg = da_ref[...] + db_ref[...] + 1.0
    h = jnp.dot(x_ref[...], w_ref[...], preferred_element_type=jnp.float32)
    g_ref[...] = h * lax.rsqrt(deg)


def _bn_body(a_ref, b_ref, g_ref, da_ref, db_ref, gam_ref, bet_ref, o_ref,
             zc_ref, st_ref):
    ph = pl.program_id(0)
    i = pl.program_id(1)

    @pl.when(jnp.logical_and(ph == 0, i == 0))
    def _():
        st_ref[...] = jnp.zeros_like(st_ref)

    @pl.when(ph == 0)
    def _():
        deg = da_ref[...] + db_ref[...] + 1.0
        z = (a_ref[...] + b_ref[...] - g_ref[...]) * lax.rsqrt(deg)
        zc_ref[pl.ds(i * _BN_BLK, _BN_BLK), :] = z
        st_ref[0:1, :] += jnp.sum(z, axis=0, keepdims=True)
        st_ref[1:2, :] += jnp.sum(z * z, axis=0, keepdims=True)

    @pl.when(ph == 1)
    def _():
        mean = st_ref[0:1, :] * (1.0 / N)
        var = st_ref[1:2, :] * (1.0 / N) - mean * mean
        rstd = lax.rsqrt(var + EPS)
        z = zc_ref[pl.ds(i * _BN_BLK, _BN_BLK), :]
        y = (z - mean) * rstd * gam_ref[...] + bet_ref[...]
        o_ref[...] = jnp.maximum(y, 0.0)


def kernel(x, edge_index, W, b, gamma, beta):
    del b
    ei = edge_index.astype(jnp.int32)

    def padded(vec, total):
        pad = N + jnp.arange(total - E, dtype=jnp.int32) % (NP - N)
        return jnp.concatenate([vec, pad])

    dst_h = padded(ei[1], EP)
    src_s = padded(ei[0], EP).reshape(EP // K, K)
    dst_s = padded(ei[1], EP).reshape(EP // K, K)

    degA2, degB2 = _hist_kernel(dst_h)
    degA = degA2.reshape(NP, 1)
    degB = degB2.reshape(NP, 1)

    x_p = jnp.pad(x, ((0, NP - N), (0, 0)))
    g = pl.pallas_call(
        _mm_body,
        grid=(NP // _MM_BLK,),
        in_specs=[
            pl.BlockSpec((_MM_BLK, D), lambda i: (i, 0)),
            pl.BlockSpec((D, D), lambda i: (0, 0)),
            pl.BlockSpec((_MM_BLK, 1), lambda i: (i, 0)),
            pl.BlockSpec((_MM_BLK, 1), lambda i: (i, 0)),
        ],
        out_specs=pl.BlockSpec((_MM_BLK, D), lambda i: (i, 0)),
        out_shape=jax.ShapeDtypeStruct((NP, D), jnp.float32),
    )(x_p, W, degA, degB)

    accA, accB = _scatter_kernel(g, src_s, dst_s)

    out = pl.pallas_call(
        _bn_body,
        grid=(2, N // _BN_BLK),
        in_specs=[
            pl.BlockSpec((_BN_BLK, D), lambda p, i: (i * (1 - p), 0)),
            pl.BlockSpec((_BN_BLK, D), lambda p, i: (i * (1 - p), 0)),
            pl.BlockSpec((_BN_BLK, D), lambda p, i: (i * (1 - p), 0)),
            pl.BlockSpec((_BN_BLK, 1), lambda p, i: (i * (1 - p), 0)),
            pl.BlockSpec((_BN_BLK, 1), lambda p, i: (i * (1 - p), 0)),
            pl.BlockSpec((1, D), lambda p, i: (0, 0)),
            pl.BlockSpec((1, D), lambda p, i: (0, 0)),
        ],
        out_specs=pl.BlockSpec((_BN_BLK, D), lambda p, i: (i, 0)),
        out_shape=jax.ShapeDtypeStruct((N, D), jnp.float32),
        scratch_shapes=[
            pltpu.VMEM((N, D), jnp.float32),
            pltpu.VMEM((8, D), jnp.float32),
        ],
    )(accA, accB, g, degA, degB,
      gamma.reshape(1, D), beta.reshape(1, D))
    return out


# speedup vs baseline: 1.0183x; 1.0183x over previous
import functools

import jax
import jax.numpy as jnp
from jax import lax
from jax.experimental import pallas as pl
from jax.experimental.pallas import tpu as pltpu
from jax.experimental.pallas import tpu_sc as plsc

N = 10000
NP = 10240
E = 320000
D = 128
EPS = 1e-5

NC = 2
NS = 16
NW = NC * NS
K = 128
CPT = 80
EP = NW * CPT * K
HCH = 40
RPW = NP // NS
HL = 16

_mesh = plsc.VectorSubcoreMesh(core_axis_name="c", subcore_axis_name="s")


NR = NP // D


@functools.partial(
    pl.kernel,
    out_type=(
        jax.ShapeDtypeStruct((NR, D), jnp.float32),
        jax.ShapeDtypeStruct((NR, D), jnp.float32),
    ),
    mesh=_mesh,
    compiler_params=pltpu.CompilerParams(needs_layout_passes=False),
    scratch_types=dict(
        deg=pltpu.VMEM_SHARED((NR, D), jnp.float32),
        hist=pltpu.VMEM((NR, D), jnp.float32),
        didxf=pltpu.VMEM((CPT * K,), jnp.int32),
        idb=pltpu.VMEM((NR,), jnp.int32),
        msem=pltpu.SemaphoreType.DMA,
    ),
)
def _hist_kernel(dst_hbm, degA_hbm, degB_hbm, *, deg, hist, didxf, idb, msem):
    c = lax.axis_index("c")
    s = lax.axis_index("s")
    wid = c * NS + s
    ebase = wid * CPT * K
    nvec = CPT * K // 16

    pltpu.sync_copy(dst_hbm.at[pl.ds(pl.multiple_of(ebase, 8), CPT * K)],
                    didxf)

    zero = jnp.zeros((16,), jnp.float32)
    ones = jnp.ones((16,), jnp.float32)
    lane = lax.iota(jnp.int32, 16)

    def fill_z(i, _):
        r = i // 8
        j = i % 8
        hist[r, pl.ds(j * 16, 16)] = zero
        return 0
    lax.fori_loop(0, NR * 8, fill_z, 0)

    def fill_id(k, _):
        idb[pl.ds(k * 16, 16)] = lane + k * 16
        return 0
    lax.fori_loop(0, NR // 16, fill_id, 0)

    def vec(i, _):
        idx = didxf[pl.ds(i * 16, 16)]
        hi = lax.shift_right_logical(idx, jnp.full((16,), 7, jnp.int32))
        lo = lax.bitwise_and(idx, jnp.full((16,), 127, jnp.int32))
        plsc.addupdate_scatter(hist, [hi, lo], ones)
        return 0
    lax.fori_loop(0, nvec, vec, 0)

    @pl.when(s == 0)
    def _():
        pltpu.sync_copy(hist, deg)
    plsc.subcore_barrier()

    @pl.when(s > 0)
    def _():
        pltpu.sync_copy(hist, deg.at[idb], add=True)
    plsc.subcore_barrier()

    @pl.when(s == 0)
    def _():
        @pl.when(c == 0)
        def _():
            pltpu.sync_copy(deg, degA_hbm)

        @pl.when(c == 1)
        def _():
            pltpu.sync_copy(deg, degB_hbm)


@functools.partial(
    pl.kernel,
    out_type=(
        jax.ShapeDtypeStruct((NP, D), jnp.float32),
        jax.ShapeDtypeStruct((NP, D), jnp.float32),
    ),
    mesh=_mesh,
    scratch_types=dict(
        acc=pltpu.VMEM_SHARED((NP, D), jnp.float32),
        rows0=pltpu.VMEM((K, D), jnp.float32),
        rows1=pltpu.VMEM((K, D), jnp.float32),
        sidx=pltpu.VMEM((HCH, K), jnp.int32),
        didx=pltpu.VMEM((HCH, K), jnp.int32),
        gsem0=pltpu.SemaphoreType.DMA,
        gsem1=pltpu.SemaphoreType.DMA,
        ssem0=pltpu.SemaphoreType.DMA,
        ssem1=pltpu.SemaphoreType.DMA,
        dsem0=pltpu.SemaphoreType.DMA,
        dsem1=pltpu.SemaphoreType.DMA,
    ),
)
def _scatter_kernel(g_hbm, src_hbm, dst_hbm, accA_hbm, accB_hbm, *,
                    acc, rows0, rows1, sidx, didx,
                    gsem0, gsem1, ssem0, ssem1, dsem0, dsem1):
    c = lax.axis_index("c")
    s = lax.axis_index("s")
    wid = c * NS + s
    cbase = wid * CPT
    row0 = s * RPW

    pltpu.sync_copy(g_hbm.at[pl.ds(row0, RPW)], acc.at[pl.ds(row0, RPW)])
    plsc.subcore_barrier()

    def pair(p, _):
        pltpu.async_copy(g_hbm.at[sidx.at[2 * p]], rows0, gsem0).wait()

        @pl.when(p > 0)
        def _():
            pltpu.make_async_copy(rows1, acc.at[didx.at[2 * p - 1]],
                                  ssem1).wait()

        s0 = pltpu.async_copy(rows0, acc.at[didx.at[2 * p]], ssem0, add=True)
        pltpu.async_copy(g_hbm.at[sidx.at[2 * p + 1]], rows1, gsem1).wait()
        s0.wait()
        pltpu.async_copy(rows1, acc.at[didx.at[2 * p + 1]], ssem1, add=True)
        return 0

    for h in range(2):
        if h > 0:
            pltpu.make_async_copy(rows1, acc.at[didx.at[HCH - 1]],
                                  ssem1).wait()
        pltpu.sync_copy(src_hbm.at[pl.ds(cbase + h * HCH, HCH)], sidx)
        pltpu.sync_copy(dst_hbm.at[pl.ds(cbase + h * HCH, HCH)], didx)
        lax.fori_loop(0, HCH // 2, pair, 0)
    pltpu.make_async_copy(rows1, acc.at[didx.at[HCH - 1]], ssem1).wait()

    plsc.subcore_barrier()

    @pl.when(c == 0)
    def _():
        pltpu.sync_copy(acc.at[pl.ds(row0, RPW)], accA_hbm.at[pl.ds(row0, RPW)])

    @pl.when(c == 1)
    def _():
        pltpu.sync_copy(acc.at[pl.ds(row0, RPW)], accB_hbm.at[pl.ds(row0, RPW)])


_MM_BLK = 1024
_BN_BLK = 1000


def _mm_body(x_ref, w_ref, da_ref, db_ref, g_ref):
    deg = da_ref[...] + db_ref[...] + 1.0
    h = jnp.dot(x_ref[...], w_ref[...], preferred_element_type=jnp.float32)
    g_ref[...] = h * lax.rsqrt(deg)


def _bn_body(a_ref, b_ref, g_ref, da_ref, db_ref, gam_ref, bet_ref, o_ref):
    deg = da_ref[...] + db_ref[...] + 1.0
    dis = lax.rsqrt(deg)
    z = (a_ref[...] + b_ref[...] - g_ref[...]) * dis
    mean = jnp.mean(z, axis=0, keepdims=True)
    zc = z - mean
    var = jnp.mean(zc * zc, axis=0, keepdims=True)
    y = zc * lax.rsqrt(var + EPS) * gam_ref[...] + bet_ref[...]
    o_ref[...] = jnp.maximum(y, 0.0)


def kernel(x, edge_index, W, b, gamma, beta):
    del b
    ei = edge_index.astype(jnp.int32)

    def padded(vec, total):
        pad = N + jnp.arange(total - E, dtype=jnp.int32) % (NP - N)
        return jnp.concatenate([vec, pad])

    dst_h = padded(ei[1], EP)
    src_s = padded(ei[0], EP).reshape(EP // K, K)
    dst_s = padded(ei[1], EP).reshape(EP // K, K)

    degA2, degB2 = _hist_kernel(dst_h)
    degA = degA2.reshape(NP, 1)
    degB = degB2.reshape(NP, 1)

    x_p = jnp.pad(x, ((0, NP - N), (0, 0)))
    g = pl.pallas_call(
        _mm_body,
        grid=(NP // _MM_BLK,),
        in_specs=[
            pl.BlockSpec((_MM_BLK, D), lambda i: (i, 0)),
            pl.BlockSpec((D, D), lambda i: (0, 0)),
            pl.BlockSpec((_MM_BLK, 1), lambda i: (i, 0)),
            pl.BlockSpec((_MM_BLK, 1), lambda i: (i, 0)),
        ],
        out_specs=pl.BlockSpec((_MM_BLK, D), lambda i: (i, 0)),
        out_shape=jax.ShapeDtypeStruct((NP, D), jnp.float32),
    )(x_p, W, degA, degB)

    accA, accB = _scatter_kernel(g, src_s, dst_s)

    out = pl.pallas_call(
        _bn_body,
        grid=(1,),
        in_specs=[
            pl.BlockSpec((N, D), lambda i: (0, 0)),
            pl.BlockSpec((N, D), lambda i: (0, 0)),
            pl.BlockSpec((N, D), lambda i: (0, 0)),
            pl.BlockSpec((N, 1), lambda i: (0, 0)),
            pl.BlockSpec((N, 1), lambda i: (0, 0)),
            pl.BlockSpec((1, D), lambda i: (0, 0)),
            pl.BlockSpec((1, D), lambda i: (0, 0)),
        ],
        out_specs=pl.BlockSpec((N, D), lambda i: (0, 0)),
        out_shape=jax.ShapeDtypeStruct((N, D), jnp.float32),
    )(accA, accB, g, degA, degB,
      gamma.reshape(1, D), beta.reshape(1, D))
    return out
